# baseline (speedup 1.0000x reference)
import functools

import jax
import jax.numpy as jnp
from jax import lax
from jax.experimental import pallas as pl
from jax.experimental.pallas import tpu as pltpu
from jax.experimental.pallas import tpu_sc as plsc

_NB = 8192
_NBC = _NB // 16
_U = 16


_G = 8


def _walk(hist, j):
    def gcond(st):
        g, fg, _ = st
        return (fg < 0) & (g < _NBC // _G)

    def gbody(st):
        g, fg, acc = st
        v = hist[pl.ds(g * _G * 16, 16)]
        for u in range(1, _G):
            v = v + hist[pl.ds((g * _G + u) * 16, 16)]
        s = jnp.sum(v)
        hit = (acc + s) >= j
        fg_new = jnp.where(hit, g, -1)
        return (g + 1, fg_new, jnp.where(hit, acc, acc + s))

    _, g, gacc = lax.while_loop(
        gcond, gbody, (jnp.int32(0), jnp.int32(-1), jnp.int32(0)))

    def cond(st):
        c, b, _, _ = st
        return (b < 0) & (c < _G)

    def body(st):
        c, b, above, acc = st
        v = hist[pl.ds((g * _G + c) * 16, 16)]
        s = jnp.sum(v)
        cs = plsc.cumsum(v)
        jrel = j - acc
        hit = (acc + s) >= j
        lane = jnp.sum(jnp.where(cs < jrel, 1, 0).astype(jnp.int32))
        wabove = jnp.sum(jnp.where(cs < jrel, v, 0))
        b_new = jnp.where(hit, (g * _G + c) * 16 + lane, -1)
        above_new = jnp.where(hit, acc + wabove, 0)
        return (c + 1, b_new, above_new, acc + s)

    _, b, above, _ = lax.while_loop(
        cond, body, (jnp.int32(0), jnp.int32(-1), jnp.int32(0), gacc))
    return b, above


@functools.lru_cache(maxsize=None)
def _build(rows, n, k):
    vecs = n // 16
    assert vecs % _U == 0
    info = plsc.get_sparse_core_info()
    nw = info.num_cores * info.num_subcores
    rpw = rows // nw
    assert rpw % 2 == 0
    pairs = rpw // 2
    mesh = plsc.VectorSubcoreMesh(core_axis_name="c", subcore_axis_name="s")

    @functools.partial(
        pl.kernel,
        mesh=mesh,
        out_type=jax.ShapeDtypeStruct((rows, n), jnp.float32),
        scratch_types=[
            pltpu.VMEM((n,), jnp.float32),
            pltpu.VMEM((n,), jnp.float32),
            pltpu.VMEM((_NB,), jnp.int32),
            pltpu.SemaphoreType.DMA,
            pltpu.SemaphoreType.DMA,
            pltpu.SemaphoreType.DMA,
            pltpu.SemaphoreType.DMA,
        ],
        compiler_params=pltpu.CompilerParams(needs_layout_passes=False),
    )
    def kern(x_hbm, out_hbm, buf_a, buf_b, hist, sin_a, sin_b, sout_a, sout_b):
        wid = lax.axis_index("s") * info.num_cores + lax.axis_index("c")
        base = wid * rpw
        zeros16 = jnp.zeros((16,), jnp.int32)
        ones16 = jnp.ones((16,), jnp.int32)

        def skey_at(buf, i):
            v = buf[pl.ds(i * 16, 16)]
            ix = lax.bitcast_convert_type(v, jnp.int32)
            return v, ix ^ ((ix >> 31) & jnp.int32(0x7FFFFFFF))

        def clear_hist():
            @plsc.parallel_loop(0, _NBC, step=1, unroll=_U)
            def _clr(c):
                hist[pl.ds(c * 16, 16)] = zeros16

        def pass1(buf):
            @plsc.parallel_loop(0, vecs, step=1, unroll=_U)
            def _p1(c):
                _, skey = skey_at(buf, c)
                rbin = 4095 - (skey >> 19)
                plsc.addupdate_scatter(hist, [rbin], ones16)

        def key2f(key):
            kv = jnp.full((16,), key, jnp.int32)
            bits = jnp.where(kv >= 0, kv, kv ^ jnp.int32(0x7FFFFFFF))
            return lax.bitcast_convert_type(bits, jnp.float32)

        def pass2(buf, lo_f, hi_f, sgn_mask):
            @plsc.parallel_loop(0, vecs, step=1, unroll=_U)
            def _p2(c):
                v = buf[pl.ds(c * 16, 16)]
                ix = lax.bitcast_convert_type(v, jnp.int32)
                pred = (v >= lo_f) & (v < hi_f)
                rbin2 = 8191 - (((ix >> 6) & jnp.int32(0x1FFF)) ^ sgn_mask)
                plsc.addupdate_scatter(hist, [rbin2], ones16, mask=pred)

        def mask_pass(buf, thr_f):
            @plsc.parallel_loop(0, vecs, step=1, unroll=_U)
            def _p3(c):
                v = buf[pl.ds(c * 16, 16)]
                buf[pl.ds(c * 16, 16)] = jnp.where(v >= thr_f, v, 0.0)

        def stage_a(buf):
            clear_hist()
            pass1(buf)
            b1, above1 = _walk(hist, jnp.int32(k))
            return b1, above1

        def stage_b(buf, b1, above1):
            clear_hist()
            t13 = 4095 - b1
            lo_f = key2f(t13 << 19)
            hi_f = jnp.where(t13 >= 4095,
                             jnp.full((16,), jnp.inf, jnp.float32),
                             key2f((t13 + 1) << 19))
            sgn_mask = jnp.full((16,), jnp.where(t13 < 0, 0x1FFF, 0),
                                jnp.int32)
            pass2(buf, lo_f, hi_f, sgn_mask)
            b2, _ = _walk(hist, jnp.int32(k) - above1)
            return (t13 << 19) | ((8191 - b2) << 6)

        def process(cur, oth, sin_cur, sout_oth, sin_oth, sout_cur,
                    row, prev_row, next_row, do_wait_out, do_start_in):
            pltpu.make_async_copy(x_hbm.at[row], cur, sin_cur).wait()
            b1, above1 = stage_a(cur)
            @pl.when(do_wait_out)
            def _():
                pltpu.make_async_copy(oth, out_hbm.at[prev_row],
                                      sout_oth).wait()

            @pl.when(do_start_in)
            def _():
                pltpu.async_copy(x_hbm.at[next_row], oth, sin_oth)
            skey_t = stage_b(cur, b1, above1)
            mask_pass(cur, key2f(skey_t))
            pltpu.async_copy(cur, out_hbm.at[row], sout_cur)

        pltpu.async_copy(x_hbm.at[base], buf_a, sin_a)

        def pair(p, _c):
            ra = base + 2 * p
            process(buf_a, buf_b, sin_a, sout_b, sin_b, sout_a,
                    ra, ra - 1, ra + 1, p >= 1, p >= 0)
            process(buf_b, buf_a, sin_b, sout_a, sin_a, sout_b,
                    ra + 1, ra, ra + 2, p >= 0, p < pairs - 1)
            return _c

        lax.fori_loop(0, pairs, pair, 0)
        pltpu.make_async_copy(buf_b, out_hbm.at[base + rpw - 1], sout_b).wait()

    return kern


def kernel(x):
    b, c, h, w = x.shape
    n = h * w
    k = int(0.05 * n)
    out = _build(b * c, n, k)(x.reshape(b * c, n))
    return out.reshape(b, c, h, w)


# speedup vs baseline: 1.0007x; 1.0007x over previous
import functools

import jax
import jax.numpy as jnp
from jax import lax
from jax.experimental import pallas as pl
from jax.experimental.pallas import tpu as pltpu
from jax.experimental.pallas import tpu_sc as plsc

_NB = 8192
_NBC = _NB // 16
_U = 16


_G = 8


def _walk(hist, j):
    def gcond(st):
        g, fg, _ = st
        return (fg < 0) & (g < _NBC // _G)

    def gbody(st):
        g, fg, acc = st
        s = jnp.int32(0)
        for u in range(_G):
            s = s + jnp.sum(hist[pl.ds((g * _G + u) * 16, 16)])
        hit = (acc + s) >= j
        fg_new = jnp.where(hit, g, -1)
        return (g + 1, fg_new, jnp.where(hit, acc, acc + s))

    _, g, gacc = lax.while_loop(
        gcond, gbody, (jnp.int32(0), jnp.int32(-1), jnp.int32(0)))

    def cond(st):
        c, b, _, _ = st
        return (b < 0) & (c < _G)

    def body(st):
        c, b, above, acc = st
        v = hist[pl.ds((g * _G + c) * 16, 16)]
        s = jnp.sum(v)
        cs = plsc.cumsum(v)
        jrel = j - acc
        hit = (acc + s) >= j
        lane = jnp.sum(jnp.where(cs < jrel, 1, 0).astype(jnp.int32))
        wabove = jnp.sum(jnp.where(cs < jrel, v, 0))
        b_new = jnp.where(hit, (g * _G + c) * 16 + lane, -1)
        above_new = jnp.where(hit, acc + wabove, 0)
        return (c + 1, b_new, above_new, acc + s)

    _, b, above, _ = lax.while_loop(
        cond, body, (jnp.int32(0), jnp.int32(-1), jnp.int32(0), gacc))
    return b, above


@functools.lru_cache(maxsize=None)
def _build(rows, n, k):
    vecs = n // 16
    assert vecs % _U == 0
    info = plsc.get_sparse_core_info()
    nw = info.num_cores * info.num_subcores
    rpw = rows // nw
    assert rpw % 2 == 0
    pairs = rpw // 2
    mesh = plsc.VectorSubcoreMesh(core_axis_name="c", subcore_axis_name="s")

    @functools.partial(
        pl.kernel,
        mesh=mesh,
        out_type=jax.ShapeDtypeStruct((rows, n), jnp.float32),
        scratch_types=[
            pltpu.VMEM((n,), jnp.float32),
            pltpu.VMEM((n,), jnp.float32),
            pltpu.VMEM((_NB,), jnp.int32),
            pltpu.SemaphoreType.DMA,
            pltpu.SemaphoreType.DMA,
            pltpu.SemaphoreType.DMA,
            pltpu.SemaphoreType.DMA,
        ],
        compiler_params=pltpu.CompilerParams(needs_layout_passes=False),
    )
    def kern(x_hbm, out_hbm, buf_a, buf_b, hist, sin_a, sin_b, sout_a, sout_b):
        wid = lax.axis_index("s") * info.num_cores + lax.axis_index("c")
        base = wid * rpw
        zeros16 = jnp.zeros((16,), jnp.int32)
        ones16 = jnp.ones((16,), jnp.int32)

        def skey_at(buf, i):
            v = buf[pl.ds(i * 16, 16)]
            ix = lax.bitcast_convert_type(v, jnp.int32)
            return v, ix ^ ((ix >> 31) & jnp.int32(0x7FFFFFFF))

        def clear_hist():
            @plsc.parallel_loop(0, _NBC, step=1, unroll=_U)
            def _clr(c):
                hist[pl.ds(c * 16, 16)] = zeros16

        def pass1(buf):
            @plsc.parallel_loop(0, vecs, step=1, unroll=_U)
            def _p1(c):
                _, skey = skey_at(buf, c)
                rbin = 4095 - (skey >> 19)
                plsc.addupdate_scatter(hist, [rbin], ones16)

        def key2f(key):
            kv = jnp.full((16,), key, jnp.int32)
            bits = jnp.where(kv >= 0, kv, kv ^ jnp.int32(0x7FFFFFFF))
            return lax.bitcast_convert_type(bits, jnp.float32)

        def pass2(buf, lo_f, hi_f, sgn_mask):
            @plsc.parallel_loop(0, vecs, step=1, unroll=_U)
            def _p2(c):
                v = buf[pl.ds(c * 16, 16)]
                ix = lax.bitcast_convert_type(v, jnp.int32)
                pred = (v >= lo_f) & (v < hi_f)
                rbin2 = 8191 - (((ix >> 6) & jnp.int32(0x1FFF)) ^ sgn_mask)
                plsc.addupdate_scatter(hist, [rbin2], ones16, mask=pred)

        def mask_pass(buf, thr_f):
            @plsc.parallel_loop(0, vecs, step=1, unroll=_U)
            def _p3(c):
                v = buf[pl.ds(c * 16, 16)]
                buf[pl.ds(c * 16, 16)] = jnp.where(v >= thr_f, v, 0.0)

        def stage_a(buf):
            clear_hist()
            pass1(buf)
            b1, above1 = _walk(hist, jnp.int32(k))
            return b1, above1

        def stage_b(buf, b1, above1):
            clear_hist()
            t13 = 4095 - b1
            lo_f = key2f(t13 << 19)
            hi_f = jnp.where(t13 >= 4095,
                             jnp.full((16,), jnp.inf, jnp.float32),
                             key2f((t13 + 1) << 19))
            sgn_mask = jnp.full((16,), jnp.where(t13 < 0, 0x1FFF, 0),
                                jnp.int32)
            pass2(buf, lo_f, hi_f, sgn_mask)
            b2, _ = _walk(hist, jnp.int32(k) - above1)
            return (t13 << 19) | ((8191 - b2) << 6)

        def process(cur, oth, sin_cur, sout_oth, sin_oth, sout_cur,
                    row, prev_row, next_row, do_wait_out, do_start_in):
            pltpu.make_async_copy(x_hbm.at[row], cur, sin_cur).wait()
            b1, above1 = stage_a(cur)
            @pl.when(do_wait_out)
            def _():
                pltpu.make_async_copy(oth, out_hbm.at[prev_row],
                                      sout_oth).wait()

            @pl.when(do_start_in)
            def _():
                pltpu.async_copy(x_hbm.at[next_row], oth, sin_oth)
            skey_t = stage_b(cur, b1, above1)
            mask_pass(cur, key2f(skey_t))
            pltpu.async_copy(cur, out_hbm.at[row], sout_cur)

        pltpu.async_copy(x_hbm.at[base], buf_a, sin_a)

        def pair(p, _c):
            ra = base + 2 * p
            process(buf_a, buf_b, sin_a, sout_b, sin_b, sout_a,
                    ra, ra - 1, ra + 1, p >= 1, p >= 0)
            process(buf_b, buf_a, sin_b, sout_a, sin_a, sout_b,
                    ra + 1, ra, ra + 2, p >= 0, p < pairs - 1)
            return _c

        lax.fori_loop(0, pairs, pair, 0)
        pltpu.make_async_copy(buf_b, out_hbm.at[base + rpw - 1], sout_b).wait()

    return kern


def kernel(x):
    b, c, h, w = x.shape
    n = h * w
    k = int(0.05 * n)
    out = _build(b * c, n, k)(x.reshape(b * c, n))
    return out.reshape(b, c, h, w)


# speedup vs baseline: 1.7202x; 1.7191x over previous
import functools

import jax
import jax.numpy as jnp
from jax import lax
from jax.experimental import pallas as pl
from jax.experimental.pallas import tpu as pltpu
from jax.experimental.pallas import tpu_sc as plsc

_NB = 8192
_NBC = _NB // 16
_U = 16


_G = 8


def _walk(hist, j):
    def gcond(st):
        g, fg, _ = st
        return (fg < 0) & (g < _NBC // _G)

    def gbody(st):
        g, fg, acc = st
        s = jnp.int32(0)
        for u in range(_G):
            s = s + jnp.sum(hist[pl.ds((g * _G + u) * 16, 16)])
        hit = (acc + s) >= j
        fg_new = jnp.where(hit, g, -1)
        return (g + 1, fg_new, jnp.where(hit, acc, acc + s))

    _, g, gacc = lax.while_loop(
        gcond, gbody, (jnp.int32(0), jnp.int32(-1), jnp.int32(0)))

    def cond(st):
        c, b, _, _ = st
        return (b < 0) & (c < _G)

    def body(st):
        c, b, above, acc = st
        v = hist[pl.ds((g * _G + c) * 16, 16)]
        s = jnp.sum(v)
        cs = plsc.cumsum(v)
        jrel = j - acc
        hit = (acc + s) >= j
        lane = jnp.sum(jnp.where(cs < jrel, 1, 0).astype(jnp.int32))
        wabove = jnp.sum(jnp.where(cs < jrel, v, 0))
        b_new = jnp.where(hit, (g * _G + c) * 16 + lane, -1)
        above_new = jnp.where(hit, acc + wabove, 0)
        return (c + 1, b_new, above_new, acc + s)

    _, b, above, _ = lax.while_loop(
        cond, body, (jnp.int32(0), jnp.int32(-1), jnp.int32(0), gacc))
    return b, above


@functools.lru_cache(maxsize=None)
def _build(rows, n, k):
    vecs = n // 16
    assert vecs % _U == 0
    info = plsc.get_sparse_core_info()
    nw = info.num_cores * info.num_subcores
    rpw = rows // nw
    assert rpw % 2 == 0
    pairs = rpw // 2
    mesh = plsc.VectorSubcoreMesh(core_axis_name="c", subcore_axis_name="s")

    @functools.partial(
        pl.kernel,
        mesh=mesh,
        out_type=jax.ShapeDtypeStruct((rows, n), jnp.float32),
        scratch_types=[
            pltpu.VMEM((n,), jnp.float32),
            pltpu.VMEM((n,), jnp.float32),
            pltpu.VMEM((_NB,), jnp.int32),
            pltpu.VMEM((16,), jnp.float32),
            pltpu.VMEM((16,), jnp.float32),
            pltpu.VMEM((16,), jnp.float32),
            pltpu.VMEM((16,), jnp.int32),
            pltpu.SemaphoreType.DMA,
            pltpu.SemaphoreType.DMA,
            pltpu.SemaphoreType.DMA,
            pltpu.SemaphoreType.DMA,
        ],
        compiler_params=pltpu.CompilerParams(needs_layout_passes=False),
    )
    def kern(x_hbm, out_hbm, buf_a, buf_b, hist, lo_r, hi_r, thr_r, sgn_r,
             sin_a, sin_b, sout_a, sout_b):
        wid = lax.axis_index("s") * info.num_cores + lax.axis_index("c")
        base = wid * rpw
        zeros16 = jnp.zeros((16,), jnp.int32)
        ones16 = jnp.ones((16,), jnp.int32)

        def skey_at(buf, i):
            v = buf[pl.ds(i * 16, 16)]
            ix = lax.bitcast_convert_type(v, jnp.int32)
            return v, ix ^ ((ix >> 31) & jnp.int32(0x7FFFFFFF))

        def clear_hist():
            @plsc.parallel_loop(0, _NBC, step=1, unroll=_U)
            def _clr(c):
                hist[pl.ds(c * 16, 16)] = zeros16

        def pass1(buf):
            @plsc.parallel_loop(0, vecs, step=1, unroll=_U)
            def _p1(c):
                _, skey = skey_at(buf, c)
                rbin = 4095 - (skey >> 19)
                plsc.addupdate_scatter(hist, [rbin], ones16)

        def key2f(key):
            kv = jnp.full((16,), key, jnp.int32)
            bits = jnp.where(kv >= 0, kv, kv ^ jnp.int32(0x7FFFFFFF))
            return lax.bitcast_convert_type(bits, jnp.float32)

        def pass2(buf):
            @plsc.parallel_loop(0, vecs, step=1, unroll=_U)
            def _p2(c):
                v = buf[pl.ds(c * 16, 16)]
                ix = lax.bitcast_convert_type(v, jnp.int32)
                pred = (v >= lo_r[...]) & (v < hi_r[...])
                rbin2 = 8191 - (((ix >> 6) & jnp.int32(0x1FFF)) ^ sgn_r[...])
                plsc.addupdate_scatter(hist, [rbin2], ones16, mask=pred)

        def mask_pass(buf):
            @plsc.parallel_loop(0, vecs, step=1, unroll=_U)
            def _p3(c):
                v = buf[pl.ds(c * 16, 16)]
                buf[pl.ds(c * 16, 16)] = jnp.where(v >= thr_r[...], v, 0.0)

        def stage_a(buf):
            clear_hist()
            pass1(buf)
            b1, above1 = _walk(hist, jnp.int32(k))
            return b1, above1

        def stage_b(buf, b1, above1):
            clear_hist()
            t13 = 4095 - b1
            lo_r[...] = key2f(t13 << 19)
            hi_r[...] = jnp.where(t13 >= 4095,
                                  jnp.full((16,), jnp.inf, jnp.float32),
                                  key2f((t13 + 1) << 19))
            sgn_r[...] = jnp.full((16,), jnp.where(t13 < 0, 0x1FFF, 0),
                                  jnp.int32)
            pass2(buf)
            b2, _ = _walk(hist, jnp.int32(k) - above1)
            return (t13 << 19) | ((8191 - b2) << 6)

        def process(cur, oth, sin_cur, sout_oth, sin_oth, sout_cur,
                    row, prev_row, next_row, do_wait_out, do_start_in):
            pltpu.make_async_copy(x_hbm.at[row], cur, sin_cur).wait()
            b1, above1 = stage_a(cur)
            @pl.when(do_wait_out)
            def _():
                pltpu.make_async_copy(oth, out_hbm.at[prev_row],
                                      sout_oth).wait()

            @pl.when(do_start_in)
            def _():
                pltpu.async_copy(x_hbm.at[next_row], oth, sin_oth)
            skey_t = stage_b(cur, b1, above1)
            thr_r[...] = key2f(skey_t)
            mask_pass(cur)
            pltpu.async_copy(cur, out_hbm.at[row], sout_cur)

        pltpu.async_copy(x_hbm.at[base], buf_a, sin_a)

        def pair(p, _c):
            ra = base + 2 * p
            process(buf_a, buf_b, sin_a, sout_b, sin_b, sout_a,
                    ra, ra - 1, ra + 1, p >= 1, p >= 0)
            process(buf_b, buf_a, sin_b, sout_a, sin_a, sout_b,
                    ra + 1, ra, ra + 2, p >= 0, p < pairs - 1)
            return _c

        lax.fori_loop(0, pairs, pair, 0)
        pltpu.make_async_copy(buf_b, out_hbm.at[base + rpw - 1], sout_b).wait()

    return kern


def kernel(x):
    b, c, h, w = x.shape
    n = h * w
    k = int(0.05 * n)
    out = _build(b * c, n, k)(x.reshape(b * c, n))
    return out.reshape(b, c, h, w)


# speedup vs baseline: 1.7360x; 1.0092x over previous
import functools

import jax
import jax.numpy as jnp
from jax import lax
from jax.experimental import pallas as pl
from jax.experimental.pallas import tpu as pltpu
from jax.experimental.pallas import tpu_sc as plsc

_NB = 8192
_NBC = _NB // 16
_U = 16


_G = 8


def _walk(hist, j):
    def gcond(st):
        g, fg, _ = st
        return (fg < 0) & (g < _NBC // _G)

    def gbody(st):
        g, fg, acc = st
        s = jnp.int32(0)
        for u in range(_G):
            s = s + jnp.sum(hist[pl.ds((g * _G + u) * 16, 16)])
        hit = (acc + s) >= j
        fg_new = jnp.where(hit, g, -1)
        return (g + 1, fg_new, jnp.where(hit, acc, acc + s))

    _, g, gacc = lax.while_loop(
        gcond, gbody, (jnp.int32(0), jnp.int32(-1), jnp.int32(0)))

    def cond(st):
        c, b, _, _ = st
        return (b < 0) & (c < _G)

    def body(st):
        c, b, above, acc = st
        v = hist[pl.ds((g * _G + c) * 16, 16)]
        s = jnp.sum(v)
        cs = plsc.cumsum(v)
        jrel = j - acc
        hit = (acc + s) >= j
        lane = jnp.sum(jnp.where(cs < jrel, 1, 0).astype(jnp.int32))
        wabove = jnp.sum(jnp.where(cs < jrel, v, 0))
        b_new = jnp.where(hit, (g * _G + c) * 16 + lane, -1)
        above_new = jnp.where(hit, acc + wabove, 0)
        return (c + 1, b_new, above_new, acc + s)

    _, b, above, _ = lax.while_loop(
        cond, body, (jnp.int32(0), jnp.int32(-1), jnp.int32(0), gacc))
    return b, above


@functools.lru_cache(maxsize=None)
def _build(rows, n, k):
    vecs = n // 16
    assert vecs % _U == 0
    info = plsc.get_sparse_core_info()
    nw = info.num_cores * info.num_subcores
    rpw = rows // nw
    assert rpw % 2 == 0
    pairs = rpw // 2
    mesh = plsc.VectorSubcoreMesh(core_axis_name="c", subcore_axis_name="s")

    @functools.partial(
        pl.kernel,
        mesh=mesh,
        out_type=jax.ShapeDtypeStruct((rows, n), jnp.float32),
        scratch_types=[
            pltpu.VMEM((n,), jnp.float32),
            pltpu.VMEM((n,), jnp.float32),
            pltpu.VMEM((_NB,), jnp.int32),
            pltpu.VMEM((16,), jnp.float32),
            pltpu.VMEM((16,), jnp.float32),
            pltpu.VMEM((16,), jnp.float32),
            pltpu.VMEM((16,), jnp.int32),
            pltpu.SemaphoreType.DMA,
            pltpu.SemaphoreType.DMA,
            pltpu.SemaphoreType.DMA,
            pltpu.SemaphoreType.DMA,
        ],
        compiler_params=pltpu.CompilerParams(needs_layout_passes=False),
    )
    def kern(x_hbm, out_hbm, buf_a, buf_b, hist, lo_r, hi_r, thr_r, sgn_r,
             sin_a, sin_b, sout_a, sout_b):
        wid = lax.axis_index("s") * info.num_cores + lax.axis_index("c")
        base = wid * rpw
        zeros16 = jnp.zeros((16,), jnp.int32)
        ones16 = jnp.ones((16,), jnp.int32)

        def skey_at(buf, i):
            v = buf[pl.ds(i * 16, 16)]
            ix = lax.bitcast_convert_type(v, jnp.int32)
            return v, ix ^ ((ix >> 31) & jnp.int32(0x7FFFFFFF))

        def clear_hist():
            @plsc.parallel_loop(0, _NBC, step=1, unroll=_U)
            def _clr(c):
                hist[pl.ds(c * 16, 16)] = zeros16

        def pass1_pos(buf):
            @plsc.parallel_loop(0, vecs, step=1, unroll=_U)
            def _p1(c):
                v = buf[pl.ds(c * 16, 16)]
                ix = lax.bitcast_convert_type(v, jnp.int32)
                rbin = 4095 - (ix >> 19)
                plsc.addupdate_scatter(hist, [rbin], ones16, mask=ix >= 0)

        def pass1_neg(buf):
            @plsc.parallel_loop(0, vecs, step=1, unroll=_U)
            def _p1n(c):
                v = buf[pl.ds(c * 16, 16)]
                ix = lax.bitcast_convert_type(v, jnp.int32)
                skey = ix ^ jnp.int32(0x7FFFFFFF)
                rbin = 4095 - (skey >> 19)
                plsc.addupdate_scatter(hist, [rbin], ones16, mask=ix < 0)

        def key2f(key):
            kv = jnp.full((16,), key, jnp.int32)
            bits = jnp.where(kv >= 0, kv, kv ^ jnp.int32(0x7FFFFFFF))
            return lax.bitcast_convert_type(bits, jnp.float32)

        def pass2(buf):
            @plsc.parallel_loop(0, vecs, step=1, unroll=_U)
            def _p2(c):
                v = buf[pl.ds(c * 16, 16)]
                ix = lax.bitcast_convert_type(v, jnp.int32)
                pred = (v >= lo_r[...]) & (v < hi_r[...])
                rbin2 = 8191 - (((ix >> 6) & jnp.int32(0x1FFF)) ^ sgn_r[...])
                plsc.addupdate_scatter(hist, [rbin2], ones16, mask=pred)

        def mask_pass(buf):
            @plsc.parallel_loop(0, vecs, step=1, unroll=_U)
            def _p3(c):
                v = buf[pl.ds(c * 16, 16)]
                buf[pl.ds(c * 16, 16)] = jnp.where(v >= thr_r[...], v, 0.0)

        def stage_a(buf):
            clear_hist()
            pass1_pos(buf)
            b1, above1 = _walk(hist, jnp.int32(k))

            def fallback(_):
                pass1_neg(buf)
                return _walk(hist, jnp.int32(k))

            b1, above1 = lax.cond(b1 < 0, fallback,
                                  lambda _: (b1, above1), 0)
            return b1, above1

        def stage_b(buf, b1, above1):
            clear_hist()
            t13 = 4095 - b1
            lo_r[...] = key2f(t13 << 19)
            hi_r[...] = jnp.where(t13 >= 4095,
                                  jnp.full((16,), jnp.inf, jnp.float32),
                                  key2f((t13 + 1) << 19))
            sgn_r[...] = jnp.full((16,), jnp.where(t13 < 0, 0x1FFF, 0),
                                  jnp.int32)
            pass2(buf)
            b2, _ = _walk(hist, jnp.int32(k) - above1)
            return (t13 << 19) | ((8191 - b2) << 6)

        def process(cur, oth, sin_cur, sout_oth, sin_oth, sout_cur,
                    row, prev_row, next_row, do_wait_out, do_start_in):
            pltpu.make_async_copy(x_hbm.at[row], cur, sin_cur).wait()
            b1, above1 = stage_a(cur)
            @pl.when(do_wait_out)
            def _():
                pltpu.make_async_copy(oth, out_hbm.at[prev_row],
                                      sout_oth).wait()

            @pl.when(do_start_in)
            def _():
                pltpu.async_copy(x_hbm.at[next_row], oth, sin_oth)
            skey_t = stage_b(cur, b1, above1)
            thr_r[...] = key2f(skey_t)
            mask_pass(cur)
            pltpu.async_copy(cur, out_hbm.at[row], sout_cur)

        pltpu.async_copy(x_hbm.at[base], buf_a, sin_a)

        def pair(p, _c):
            ra = base + 2 * p
            process(buf_a, buf_b, sin_a, sout_b, sin_b, sout_a,
                    ra, ra - 1, ra + 1, p >= 1, p >= 0)
            process(buf_b, buf_a, sin_b, sout_a, sin_a, sout_b,
                    ra + 1, ra, ra + 2, p >= 0, p < pairs - 1)
            return _c

        lax.fori_loop(0, pairs, pair, 0)
        pltpu.make_async_copy(buf_b, out_hbm.at[base + rpw - 1], sout_b).wait()

    return kern


def kernel(x):
    b, c, h, w = x.shape
    n = h * w
    k = int(0.05 * n)
    out = _build(b * c, n, k)(x.reshape(b * c, n))
    return out.reshape(b, c, h, w)
